# baseline (speedup 1.0000x reference)
import functools

import jax
import jax.numpy as jnp
from jax import lax
from jax.experimental import pallas as pl
from jax.experimental.pallas import tpu as pltpu
from jax.experimental.pallas import tpu_sc as plsc

_BATCH = 16384
_DIM = 32
_NW = 32
_B_PER_W = _BATCH // _NW
_CHUNK = 128
_NCHUNK = _B_PER_W // _CHUNK


def _sc_gather(r_idx2d, Wu, rvh):
    mesh = plsc.VectorSubcoreMesh(core_axis_name="c", subcore_axis_name="s")

    @functools.partial(
        pl.kernel,
        out_type=(
            jax.ShapeDtypeStruct((_BATCH, _DIM), jnp.float32),
            jax.ShapeDtypeStruct((_BATCH, _DIM), jnp.float32),
        ),
        mesh=mesh,
        scratch_types=[
            pltpu.VMEM((_NCHUNK, _CHUNK), jnp.int32),
            pltpu.VMEM((_B_PER_W, _DIM), jnp.float32),
            pltpu.VMEM((_B_PER_W, _DIM), jnp.float32),
            pltpu.SemaphoreType.DMA,
        ],
        compiler_params=pltpu.CompilerParams(use_tc_tiling_on_sc=False),
    )
    def gather_kernel(idx_hbm, wu_hbm, rvh_hbm, ru_out, rv_out,
                      idx_v, ru_v, rv_v, sem):
        wid = lax.axis_index("s") * 2 + lax.axis_index("c")
        base = wid * _B_PER_W
        pltpu.sync_copy(idx_hbm.at[pl.ds(wid * _NCHUNK, _NCHUNK)], idx_v)
        copies = []
        for j in range(_NCHUNK):
            copies.append(pltpu.async_copy(
                wu_hbm.at[idx_v.at[j]],
                ru_v.at[pl.ds(j * _CHUNK, _CHUNK)], sem))
            copies.append(pltpu.async_copy(
                rvh_hbm.at[idx_v.at[j]],
                rv_v.at[pl.ds(j * _CHUNK, _CHUNK)], sem))
        for c in copies:
            c.wait()
        pltpu.sync_copy(ru_v, ru_out.at[pl.ds(base, _B_PER_W)])
        pltpu.sync_copy(rv_v, rv_out.at[pl.ds(base, _B_PER_W)])

    return gather_kernel(r_idx2d, Wu, rvh)


def _artanh(x):
    return 0.5 * jnp.log((1.0 + x) / (1.0 - x))


def _rownorm(x):
    return jnp.sqrt(jnp.sum(x * x, axis=-1, keepdims=True))


def _proj_rows(e):
    n = _rownorm(e)
    return jnp.where(n >= 1.0, e / (n - 1e-05), e)


def _p_sum(x, y):
    sqxnorm = jnp.sum(x * x, axis=-1, keepdims=True)
    sqynorm = jnp.sum(y * y, axis=-1, keepdims=True)
    dotxy = jnp.sum(x * y, axis=-1, keepdims=True)
    numerator = (1.0 + 2.0 * dotxy + sqynorm) * x + (1.0 - sqxnorm) * y
    denominator = 1.0 + 2.0 * dotxy + sqxnorm * sqynorm
    return numerator / denominator


def _math_body(u_ref, v_ref, ru_ref, rv_ref, out_ref):
    u = _proj_rows(u_ref[...])
    v = _proj_rows(v_ref[...])
    Ru = ru_ref[...]
    rv = _proj_rows(rv_ref[...])
    un = jnp.clip(_rownorm(u), 1e-10, 1.0 - 1e-05)
    u_e = _artanh(un) / un * u
    u_W = u_e * Ru
    wn = jnp.maximum(_rownorm(u_W), 1e-10)
    u_m = jnp.tanh(wn) / wn * u_W
    v_m = _p_sum(v, rv)
    u_m = _proj_rows(u_m)
    v_m = _proj_rows(v_m)
    diff = _p_sum(-u_m, v_m)
    diff_norm = jnp.clip(_rownorm(diff), 1e-10, 1.0 - 1e-05)
    sqdist = (2.0 * _artanh(diff_norm)) ** 2
    out_ref[...] = -sqdist


def _tc_math(u_emb, v_emb, Ru, Rv, block_rows=2048):
    grid = _BATCH // block_rows
    row_spec = pl.BlockSpec((block_rows, _DIM), lambda i: (i, 0))
    return pl.pallas_call(
        _math_body,
        grid=(grid,),
        in_specs=[row_spec, row_spec, row_spec, row_spec],
        out_specs=pl.BlockSpec((block_rows, 1), lambda i: (i, 0)),
        out_shape=jax.ShapeDtypeStruct((_BATCH, 1), jnp.float32),
    )(u_emb, v_emb, Ru, Rv)


def kernel(u_emb, r_idx, v_emb, Wu, rvh):
    r_idx2d = r_idx.reshape(_NW * _NCHUNK, _CHUNK)
    Ru, Rv = _sc_gather(r_idx2d, Wu, rvh)
    score = _tc_math(u_emb, v_emb, Ru, Rv)
    return score.reshape(_BATCH)
